# baseline (speedup 1.0000x reference)
import jax
import jax.numpy as jnp
from jax.experimental import pallas as pl

N = 8192
ROWS = 4096
BLK = 256
WEIGHT = 0.999


def _hist_kernel(mask_ref, p_ref, out_ref):
    i = pl.program_id(0)
    m = mask_ref[...].astype(jnp.int32)
    lengths = jnp.sum(m, axis=1, keepdims=True)
    bins = jax.lax.broadcasted_iota(jnp.int32, (1, N), 1) + 1
    part = jnp.sum((lengths == bins).astype(jnp.float32), axis=0, keepdims=True)

    @pl.when(i == 0)
    def _init():
        out_ref[...] = jnp.zeros_like(out_ref)

    out_ref[...] += part

    @pl.when(i == pl.num_programs(0) - 1)
    def _finish():
        out_ref[...] = WEIGHT * p_ref[...] + ((1.0 - WEIGHT) / ROWS) * out_ref[...]


def kernel(mask, n_elements_prob):
    p2 = n_elements_prob.reshape(1, N)
    out = pl.pallas_call(
        _hist_kernel,
        grid=(ROWS // BLK,),
        in_specs=[
            pl.BlockSpec((BLK, N), lambda i: (i, 0)),
            pl.BlockSpec((1, N), lambda i: (0, 0)),
        ],
        out_specs=pl.BlockSpec((1, N), lambda i: (0, 0)),
        out_shape=jax.ShapeDtypeStruct((1, N), jnp.float32),
    )(mask, p2)
    return out.reshape(N)
